# baseline (speedup 1.0000x reference)
import functools

import jax
import jax.numpy as jnp
from jax import lax
from jax.experimental import pallas as pl
from jax.experimental.pallas import tpu as pltpu
from jax.experimental.pallas import tpu_sc as plsc

GAMMA = 0.01
INV_GAMMA = 1.0 / GAMMA
LN2 = 0.6931471805599453
C, G, S, L = 32, 4096, 16, 3
B, M = 32, 4
NC, NS, LANES = 2, 16, 16
NW = NC * NS
B2 = B // NC
GSL = G // NS
NG16 = GSL // LANES
SL = S * L
SH = S // 2
CHUNK = SL * GSL
PER = B2 * GSL
CL = C * LANES
NEG_BIG = -1e30


def _ln(v):
    bits = lax.bitcast_convert_type(v, jnp.int32)
    e = lax.shift_right_arithmetic(bits, 23) - 127
    m = lax.bitcast_convert_type(
        lax.bitwise_or(lax.bitwise_and(bits, 0x7FFFFF), 0x3F800000),
        jnp.float32)
    r = (m - 1.0) / (m + 1.0)
    r2 = r * r
    p = r * (2.0 + r2 * (0.66666667 + r2 * (0.4 + r2 * 0.28571429)))
    return e.astype(jnp.float32) * LN2 + p


def _sc_clauses_body(x_hbm, it_hbm, y_hbm, tmax_hbm,
                     table, idxbuf, pmx, psum, yslab, maxbuf, dsem):
    cid = lax.axis_index("c")
    sid = lax.axis_index("s")
    wid = sid * NC + cid

    pltpu.sync_copy(x_hbm.at[pl.ds(cid * (B2 * G), B2 * G)], table)
    pltpu.async_copy(it_hbm.at[0, sid], idxbuf.at[pl.ds(0, CHUNK)], dsem)

    def clause_body(c, carry):
        slot = lax.rem(c, 2) * CHUNK
        pltpu.make_async_copy(
            it_hbm.at[c, sid], idxbuf.at[pl.ds(slot, CHUNK)], dsem).wait()

        @pl.when(c + 1 < C)
        def _start_next():
            pltpu.async_copy(
                it_hbm.at[c + 1, sid],
                idxbuf.at[pl.ds(CHUNK - slot, CHUNK)], dsem)

        def g16_body(g16, cmax):
            base = g16 * LANES

            def halve(off):
                idx = [idxbuf[pl.ds(slot + (off * L * SH + j) * GSL + base,
                                    LANES)]
                       for j in range(L * SH)]

                def soft_partial(b):
                    row = table.at[pl.ds(b * G, G)]
                    prods = []
                    for s in range(SH):
                        g0 = plsc.load_gather(row, [idx[s * L]])
                        g1 = plsc.load_gather(row, [idx[s * L + 1]])
                        g2 = plsc.load_gather(row, [idx[s * L + 2]])
                        prods.append(g0 * g1 * g2)
                    mx = prods[0]
                    for s in range(1, SH):
                        mx = jnp.maximum(mx, prods[s])
                    se = jnp.exp((prods[0] - mx) * INV_GAMMA)
                    for s in range(1, SH):
                        se = se + jnp.exp((prods[s] - mx) * INV_GAMMA)
                    return mx, se
                return soft_partial

            part_a = halve(0)

            def body_a(b, t):
                mx, se = part_a(b)
                pmx[pl.ds(b * LANES, LANES)] = mx
                psum[pl.ds(b * LANES, LANES)] = se
                return t

            lax.fori_loop(0, B2, body_a, jnp.int32(0), unroll=2)

            part_b = halve(1)

            def body_b(b, cm):
                mxb, seb = part_b(b)
                mxa = pmx[pl.ds(b * LANES, LANES)]
                sea = psum[pl.ds(b * LANES, LANES)]
                mx = jnp.maximum(mxa, mxb)
                se = (sea * jnp.exp((mxa - mx) * INV_GAMMA)
                      + seb * jnp.exp((mxb - mx) * INV_GAMMA))
                y = mx + GAMMA * _ln(se)
                yslab[pl.ds(b * GSL + base, LANES)] = y
                return jnp.maximum(cm, y)

            return lax.fori_loop(0, B2, body_b, cmax, unroll=2)

        cmax0 = jnp.full((LANES,), NEG_BIG, jnp.float32)
        cmax = lax.fori_loop(0, NG16, g16_body, cmax0, unroll=False)
        maxbuf[pl.ds(c * LANES, LANES)] = cmax
        pltpu.sync_copy(yslab, y_hbm.at[sid, c, cid])
        return carry

    lax.fori_loop(0, C, clause_body, jnp.int32(0), unroll=False)
    pltpu.sync_copy(maxbuf, tmax_hbm.at[wid])


@functools.cache
def _sc_clauses():
    mesh = plsc.VectorSubcoreMesh(
        core_axis_name="c", subcore_axis_name="s",
        num_cores=NC, num_subcores=NS)
    return pl.kernel(
        _sc_clauses_body,
        out_type=[
            jax.ShapeDtypeStruct((NS, C, NC, PER), jnp.float32),
            jax.ShapeDtypeStruct((NW, CL), jnp.float32),
        ],
        mesh=mesh,
        compiler_params=pltpu.CompilerParams(needs_layout_passes=False),
        scratch_types=[
            pltpu.VMEM((B2 * G,), jnp.float32),
            pltpu.VMEM((2 * CHUNK,), jnp.int32),
            pltpu.VMEM((B2 * LANES,), jnp.float32),
            pltpu.VMEM((B2 * LANES,), jnp.float32),
            pltpu.VMEM((PER,), jnp.float32),
            pltpu.VMEM((CL,), jnp.float32),
            pltpu.SemaphoreType.DMA,
        ],
    )


def _tc_merge_body(r_ref, y_ref, tmax_ref, w_ref, out_ref, yh_ref, mh_ref):
    i = pl.program_id(0)

    wv = w_ref[...]
    wmx = jnp.max(wv, axis=1, keepdims=True)
    we = jnp.exp(wv - wmx)
    wstar = we / jnp.sum(we, axis=1, keepdims=True)
    mxc = jnp.max(tmax_ref[...].reshape(NW, C, LANES), axis=(0, 2))
    alpha = jnp.where(mxc > 1.0, 1.0 / mxc, 1.0)
    amat = wstar * alpha[None, :]

    @pl.when(i < NS)
    def _chunk():
        yb = y_ref[...]
        cms = []
        for cid in range(NC):
            yc = yb[0, :, cid, :]
            h = jnp.dot(amat, yc, preferred_element_type=jnp.float32)
            hm = jnp.max(h, axis=0, keepdims=True)
            se = jnp.sum(jnp.exp((h - hm) * INV_GAMMA), axis=0, keepdims=True)
            yh = (hm + GAMMA * jnp.log(se)).reshape(B2, GSL)
            yh_ref[pl.ds(cid * B2, B2), pl.ds(i * GSL, GSL)] = yh
            cms.append(jnp.max(yh))
        cm = jnp.maximum(cms[0], cms[1])

        @pl.when(i == 0)
        def _():
            mh_ref[0] = cm

        @pl.when(i > 0)
        def _():
            mh_ref[0] = jnp.maximum(mh_ref[0], cm)

    @pl.when(i == NS)
    def _finish():
        mh = mh_ref[0]
        beta = jnp.where(mh > 1.0, 1.0 / mh, 1.0)
        r = yh_ref[...] * beta
        rv = r_ref[...]
        mx2 = jnp.maximum(rv, r)
        mn2 = jnp.minimum(rv, r)
        z = mx2 + GAMMA * jnp.log(1.0 + jnp.exp((mn2 - mx2) * INV_GAMMA))
        mz = jnp.max(z)
        out_ref[...] = jnp.where(mz > 1.0, z / mz, z)


_tc_merge = pl.pallas_call(
    _tc_merge_body,
    grid=(NS + 1,),
    in_specs=[
        pl.BlockSpec((B, G), lambda i: (0, 0)),
        pl.BlockSpec((1, C, NC, PER),
                     lambda i: (jnp.minimum(i, NS - 1), 0, 0, 0)),
        pl.BlockSpec((NW, CL), lambda i: (0, 0)),
        pl.BlockSpec((M, C), lambda i: (0, 0)),
    ],
    out_specs=pl.BlockSpec((B, G), lambda i: (0, 0)),
    out_shape=jax.ShapeDtypeStruct((B, G), jnp.float32),
    scratch_shapes=[
        pltpu.VMEM((B, G), jnp.float32),
        pltpu.SMEM((1,), jnp.float32),
    ],
)


def kernel(x, W, I):
    it2 = (jnp.transpose(I, (0, 2, 3, 1))
           .reshape(C, SL, NS, GSL)
           .transpose(0, 2, 1, 3)
           .reshape(C, NS, CHUNK))
    r = x
    for _ in range(2):
        y, tmax = _sc_clauses()(r.reshape(B * G), it2)
        r = _tc_merge(r, y, tmax, W)
    return r


# speedup vs baseline: 1.3137x; 1.3137x over previous
import functools

import jax
import jax.numpy as jnp
from jax import lax
from jax.experimental import pallas as pl
from jax.experimental.pallas import tpu as pltpu
from jax.experimental.pallas import tpu_sc as plsc

GAMMA = 0.01
INV_GAMMA = 1.0 / GAMMA
LN2 = 0.6931471805599453
C, G, S, L = 32, 4096, 16, 3
B, M = 32, 4
NC, NS, LANES = 2, 16, 16
NW = NC * NS
B2 = B // NC
GSL = G // NS
NG16 = GSL // LANES
SL = S * L
SH = S // 2
CHUNK = SL * GSL
PER = B2 * GSL
CL = C * LANES
NEG_BIG = -1e30


def _ln(v):
    bits = lax.bitcast_convert_type(v, jnp.int32)
    e = lax.shift_right_arithmetic(bits, 23) - 127
    m = lax.bitcast_convert_type(
        lax.bitwise_or(lax.bitwise_and(bits, 0x7FFFFF), 0x3F800000),
        jnp.float32)
    r = (m - 1.0) / (m + 1.0)
    r2 = r * r
    p = r * (2.0 + r2 * (0.66666667 + r2 * (0.4 + r2 * 0.28571429)))
    return e.astype(jnp.float32) * LN2 + p


NP = B2 // 2


def _unpk(w):
    lo = lax.bitcast_convert_type(lax.shift_left(w, 16), jnp.float32)
    hi = lax.bitcast_convert_type(lax.bitwise_and(w, -65536), jnp.float32)
    return lo, hi


def _sc_clauses_body(x_hbm, it_hbm, y_hbm, tmax_hbm,
                     table, idxbuf, pmx, psum, yslab, maxbuf, dsem):
    cid = lax.axis_index("c")
    sid = lax.axis_index("s")
    wid = sid * NC + cid

    pltpu.sync_copy(x_hbm.at[pl.ds(cid * (NP * G), NP * G)], table)
    pltpu.async_copy(it_hbm.at[0, sid], idxbuf.at[pl.ds(0, CHUNK)], dsem)

    def clause_body(c, carry):
        slot = lax.rem(c, 2) * CHUNK
        pltpu.make_async_copy(
            it_hbm.at[c, sid], idxbuf.at[pl.ds(slot, CHUNK)], dsem).wait()

        @pl.when(c + 1 < C)
        def _start_next():
            pltpu.async_copy(
                it_hbm.at[c + 1, sid],
                idxbuf.at[pl.ds(CHUNK - slot, CHUNK)], dsem)

        def g16_body(g16, cmax):
            base = g16 * LANES

            def halve(off):
                idx = [idxbuf[pl.ds(slot + (off * L * SH + j) * GSL + base,
                                    LANES)]
                       for j in range(L * SH)]

                def soft_partial(p):
                    row = table.at[pl.ds(p * G, G)]
                    pr0, pr1 = [], []
                    for s in range(SH):
                        w0 = plsc.load_gather(row, [idx[s * L]])
                        w1 = plsc.load_gather(row, [idx[s * L + 1]])
                        w2 = plsc.load_gather(row, [idx[s * L + 2]])
                        a0, a1 = _unpk(w0)
                        b0, b1 = _unpk(w1)
                        c0, c1 = _unpk(w2)
                        pr0.append(a0 * b0 * c0)
                        pr1.append(a1 * b1 * c1)
                    out = []
                    for pr in (pr0, pr1):
                        mx = pr[0]
                        for s in range(1, SH):
                            mx = jnp.maximum(mx, pr[s])
                        se = jnp.exp((pr[0] - mx) * INV_GAMMA)
                        for s in range(1, SH):
                            se = se + jnp.exp((pr[s] - mx) * INV_GAMMA)
                        out.append((mx, se))
                    return out
                return soft_partial

            part_a = halve(0)

            def body_a(p, t):
                for k, (mx, se) in enumerate(part_a(p)):
                    pmx[pl.ds((2 * p + k) * LANES, LANES)] = mx
                    psum[pl.ds((2 * p + k) * LANES, LANES)] = se
                return t

            lax.fori_loop(0, NP, body_a, jnp.int32(0), unroll=False)

            part_b = halve(1)

            def body_b(p, cm):
                for k, (mxb, seb) in enumerate(part_b(p)):
                    mxa = pmx[pl.ds((2 * p + k) * LANES, LANES)]
                    sea = psum[pl.ds((2 * p + k) * LANES, LANES)]
                    mx = jnp.maximum(mxa, mxb)
                    se = (sea * jnp.exp((mxa - mx) * INV_GAMMA)
                          + seb * jnp.exp((mxb - mx) * INV_GAMMA))
                    y = mx + GAMMA * _ln(se)
                    yslab[pl.ds((2 * p + k) * GSL + base, LANES)] = y
                    cm = jnp.maximum(cm, y)
                return cm

            return lax.fori_loop(0, NP, body_b, cmax, unroll=False)

        cmax0 = jnp.full((LANES,), NEG_BIG, jnp.float32)
        cmax = lax.fori_loop(0, NG16, g16_body, cmax0, unroll=False)
        maxbuf[pl.ds(c * LANES, LANES)] = cmax
        pltpu.sync_copy(yslab, y_hbm.at[sid, c, cid])
        return carry

    lax.fori_loop(0, C, clause_body, jnp.int32(0), unroll=False)
    pltpu.sync_copy(maxbuf, tmax_hbm.at[wid])


@functools.cache
def _sc_clauses():
    mesh = plsc.VectorSubcoreMesh(
        core_axis_name="c", subcore_axis_name="s",
        num_cores=NC, num_subcores=NS)
    return pl.kernel(
        _sc_clauses_body,
        out_type=[
            jax.ShapeDtypeStruct((NS, C, NC, PER), jnp.float32),
            jax.ShapeDtypeStruct((NW, CL), jnp.float32),
        ],
        mesh=mesh,
        compiler_params=pltpu.CompilerParams(needs_layout_passes=False),
        scratch_types=[
            pltpu.VMEM((NP * G,), jnp.int32),
            pltpu.VMEM((2 * CHUNK,), jnp.int32),
            pltpu.VMEM((B2 * LANES,), jnp.float32),
            pltpu.VMEM((B2 * LANES,), jnp.float32),
            pltpu.VMEM((PER,), jnp.float32),
            pltpu.VMEM((CL,), jnp.float32),
            pltpu.SemaphoreType.DMA,
        ],
    )


def _tc_merge_body(r_ref, y_ref, tmax_ref, w_ref, out_ref, yh_ref, mh_ref):
    i = pl.program_id(0)

    wv = w_ref[...]
    wmx = jnp.max(wv, axis=1, keepdims=True)
    we = jnp.exp(wv - wmx)
    wstar = we / jnp.sum(we, axis=1, keepdims=True)
    mxc = jnp.max(tmax_ref[...].reshape(NW, C, LANES), axis=(0, 2))
    alpha = jnp.where(mxc > 1.0, 1.0 / mxc, 1.0)
    amat = wstar * alpha[None, :]

    @pl.when(i < NS)
    def _chunk():
        yb = y_ref[...]
        cms = []
        for cid in range(NC):
            yc = yb[0, :, cid, :]
            h = jnp.dot(amat, yc, preferred_element_type=jnp.float32)
            hm = jnp.max(h, axis=0, keepdims=True)
            se = jnp.sum(jnp.exp((h - hm) * INV_GAMMA), axis=0, keepdims=True)
            yh = (hm + GAMMA * jnp.log(se)).reshape(B2, GSL)
            yh_ref[pl.ds(cid * B2, B2), pl.ds(i * GSL, GSL)] = yh
            cms.append(jnp.max(yh))
        cm = jnp.maximum(cms[0], cms[1])

        @pl.when(i == 0)
        def _():
            mh_ref[0] = cm

        @pl.when(i > 0)
        def _():
            mh_ref[0] = jnp.maximum(mh_ref[0], cm)

    @pl.when(i == NS)
    def _finish():
        mh = mh_ref[0]
        beta = jnp.where(mh > 1.0, 1.0 / mh, 1.0)
        r = yh_ref[...] * beta
        rv = r_ref[...]
        mx2 = jnp.maximum(rv, r)
        mn2 = jnp.minimum(rv, r)
        z = mx2 + GAMMA * jnp.log(1.0 + jnp.exp((mn2 - mx2) * INV_GAMMA))
        mz = jnp.max(z)
        out_ref[...] = jnp.where(mz > 1.0, z / mz, z)


_tc_merge = pl.pallas_call(
    _tc_merge_body,
    grid=(NS + 1,),
    in_specs=[
        pl.BlockSpec((B, G), lambda i: (0, 0)),
        pl.BlockSpec((1, C, NC, PER),
                     lambda i: (jnp.minimum(i, NS - 1), 0, 0, 0)),
        pl.BlockSpec((NW, CL), lambda i: (0, 0)),
        pl.BlockSpec((M, C), lambda i: (0, 0)),
    ],
    out_specs=pl.BlockSpec((B, G), lambda i: (0, 0)),
    out_shape=jax.ShapeDtypeStruct((B, G), jnp.float32),
    scratch_shapes=[
        pltpu.VMEM((B, G), jnp.float32),
        pltpu.SMEM((1,), jnp.float32),
    ],
)


def kernel(x, W, I):
    it2 = (jnp.transpose(I, (0, 2, 3, 1))
           .reshape(C, SL, NS, GSL)
           .transpose(0, 2, 1, 3)
           .reshape(C, NS, CHUNK))
    r = x
    for _ in range(2):
        u = lax.bitcast_convert_type(
            lax.convert_element_type(r, jnp.bfloat16), jnp.uint16)
        pk = lax.bitcast_convert_type(
            u[0::2].astype(jnp.uint32) | (u[1::2].astype(jnp.uint32) << 16),
            jnp.int32)
        y, tmax = _sc_clauses()(pk.reshape(B // 2 * G), it2)
        r = _tc_merge(r, y, tmax, W)
    return r


# speedup vs baseline: 1.3244x; 1.0082x over previous
import functools

import jax
import jax.numpy as jnp
from jax import lax
from jax.experimental import pallas as pl
from jax.experimental.pallas import tpu as pltpu
from jax.experimental.pallas import tpu_sc as plsc

GAMMA = 0.01
INV_GAMMA = 1.0 / GAMMA
LN2 = 0.6931471805599453
C, G, S, L = 32, 4096, 16, 3
B, M = 32, 4
NC, NS, LANES = 2, 16, 16
NW = NC * NS
B2 = B // NC
GSL = G // NS
NG16 = GSL // LANES
SL = S * L
SH = S // 2
CHUNK = SL * GSL
PER = B2 * GSL
CL = C * LANES
NEG_BIG = -1e30


def _ln(v):
    bits = lax.bitcast_convert_type(v, jnp.int32)
    e = lax.shift_right_arithmetic(bits, 23) - 127
    m = lax.bitcast_convert_type(
        lax.bitwise_or(lax.bitwise_and(bits, 0x7FFFFF), 0x3F800000),
        jnp.float32)
    r = (m - 1.0) / (m + 1.0)
    r2 = r * r
    p = r * (2.0 + r2 * (0.66666667 + r2 * (0.4 + r2 * 0.28571429)))
    return e.astype(jnp.float32) * LN2 + p


NP = B2 // 2


def _unpk(w):
    lo = lax.bitcast_convert_type(lax.shift_left(w, 16), jnp.float32)
    hi = lax.bitcast_convert_type(lax.bitwise_and(w, -65536), jnp.float32)
    return lo, hi


def _sc_clauses_body(x_hbm, it_hbm, y_hbm, tmax_hbm,
                     table, idxbuf, pmx, psum, yslab, maxbuf, dsem, ysem):
    cid = lax.axis_index("c")
    sid = lax.axis_index("s")
    wid = sid * NC + cid

    pltpu.sync_copy(x_hbm.at[pl.ds(cid * (NP * G), NP * G)], table)
    pltpu.async_copy(it_hbm.at[0, sid], idxbuf.at[pl.ds(0, CHUNK)], dsem)

    def clause_body(c, carry):
        slot = lax.rem(c, 2) * CHUNK
        pltpu.make_async_copy(
            it_hbm.at[c, sid], idxbuf.at[pl.ds(slot, CHUNK)], dsem).wait()

        @pl.when(c + 1 < C)
        def _start_next():
            pltpu.async_copy(
                it_hbm.at[c + 1, sid],
                idxbuf.at[pl.ds(CHUNK - slot, CHUNK)], dsem)

        def g16_body(g16, cmax):
            base = g16 * LANES

            def halve(off):
                idx = [idxbuf[pl.ds(slot + (off * L * SH + j) * GSL + base,
                                    LANES)]
                       for j in range(L * SH)]

                def soft_partial(p):
                    row = table.at[pl.ds(p * G, G)]
                    pr0, pr1 = [], []
                    for s in range(SH):
                        w0 = plsc.load_gather(row, [idx[s * L]])
                        w1 = plsc.load_gather(row, [idx[s * L + 1]])
                        w2 = plsc.load_gather(row, [idx[s * L + 2]])
                        a0, a1 = _unpk(w0)
                        b0, b1 = _unpk(w1)
                        c0, c1 = _unpk(w2)
                        pr0.append(a0 * b0 * c0)
                        pr1.append(a1 * b1 * c1)
                    out = []
                    for pr in (pr0, pr1):
                        mx = pr[0]
                        for s in range(1, SH):
                            mx = jnp.maximum(mx, pr[s])
                        se = jnp.exp((pr[0] - mx) * INV_GAMMA)
                        for s in range(1, SH):
                            se = se + jnp.exp((pr[s] - mx) * INV_GAMMA)
                        out.append((mx, se))
                    return out
                return soft_partial

            part_a = halve(0)

            def body_a(p, t):
                for k, (mx, se) in enumerate(part_a(p)):
                    pmx[pl.ds((2 * p + k) * LANES, LANES)] = mx
                    psum[pl.ds((2 * p + k) * LANES, LANES)] = se
                return t

            lax.fori_loop(0, NP, body_a, jnp.int32(0), unroll=False)

            part_b = halve(1)

            def body_b(p, cm):
                for k, (mxb, seb) in enumerate(part_b(p)):
                    mxa = pmx[pl.ds((2 * p + k) * LANES, LANES)]
                    sea = psum[pl.ds((2 * p + k) * LANES, LANES)]
                    mx = jnp.maximum(mxa, mxb)
                    se = (sea * jnp.exp((mxa - mx) * INV_GAMMA)
                          + seb * jnp.exp((mxb - mx) * INV_GAMMA))
                    y = mx + GAMMA * _ln(se)
                    yslab[pl.ds(yslot + (2 * p + k) * GSL + base, LANES)] = y
                    cm = jnp.maximum(cm, y)
                return cm

            return lax.fori_loop(0, NP, body_b, cmax, unroll=False)

        yslot = lax.rem(c, 2) * PER

        @pl.when(c >= 2)
        def _drain_prev():
            pltpu.make_async_copy(
                yslab.at[pl.ds(yslot, PER)], y_hbm.at[sid, c - 2, cid],
                ysem).wait()

        cmax0 = jnp.full((LANES,), NEG_BIG, jnp.float32)
        cmax = lax.fori_loop(0, NG16, g16_body, cmax0, unroll=False)
        maxbuf[pl.ds(c * LANES, LANES)] = cmax
        pltpu.async_copy(
            yslab.at[pl.ds(yslot, PER)], y_hbm.at[sid, c, cid], ysem)
        return carry

    lax.fori_loop(0, C, clause_body, jnp.int32(0), unroll=False)
    pltpu.make_async_copy(
        yslab.at[pl.ds(0, PER)], y_hbm.at[sid, C - 2, cid], ysem).wait()
    pltpu.make_async_copy(
        yslab.at[pl.ds(PER, PER)], y_hbm.at[sid, C - 1, cid], ysem).wait()
    pltpu.sync_copy(maxbuf, tmax_hbm.at[wid])


@functools.cache
def _sc_clauses():
    mesh = plsc.VectorSubcoreMesh(
        core_axis_name="c", subcore_axis_name="s",
        num_cores=NC, num_subcores=NS)
    return pl.kernel(
        _sc_clauses_body,
        out_type=[
            jax.ShapeDtypeStruct((NS, C, NC, PER), jnp.float32),
            jax.ShapeDtypeStruct((NW, CL), jnp.float32),
        ],
        mesh=mesh,
        compiler_params=pltpu.CompilerParams(needs_layout_passes=False),
        scratch_types=[
            pltpu.VMEM((NP * G,), jnp.int32),
            pltpu.VMEM((2 * CHUNK,), jnp.int32),
            pltpu.VMEM((B2 * LANES,), jnp.float32),
            pltpu.VMEM((B2 * LANES,), jnp.float32),
            pltpu.VMEM((2 * PER,), jnp.float32),
            pltpu.VMEM((CL,), jnp.float32),
            pltpu.SemaphoreType.DMA,
            pltpu.SemaphoreType.DMA,
        ],
    )


def _tc_merge_body(r_ref, y_ref, tmax_ref, w_ref, out_ref, yh_ref, mh_ref):
    i = pl.program_id(0)

    wv = w_ref[...]
    wmx = jnp.max(wv, axis=1, keepdims=True)
    we = jnp.exp(wv - wmx)
    wstar = we / jnp.sum(we, axis=1, keepdims=True)
    mxc = jnp.max(tmax_ref[...].reshape(NW, C, LANES), axis=(0, 2))
    alpha = jnp.where(mxc > 1.0, 1.0 / mxc, 1.0)
    amat = wstar * alpha[None, :]

    @pl.when(i < NS)
    def _chunk():
        yb = y_ref[...]
        cms = []
        for cid in range(NC):
            yc = yb[0, :, cid, :]
            h = jnp.dot(amat, yc, preferred_element_type=jnp.float32)
            hm = jnp.max(h, axis=0, keepdims=True)
            se = jnp.sum(jnp.exp((h - hm) * INV_GAMMA), axis=0, keepdims=True)
            yh = (hm + GAMMA * jnp.log(se)).reshape(B2, GSL)
            yh_ref[pl.ds(cid * B2, B2), pl.ds(i * GSL, GSL)] = yh
            cms.append(jnp.max(yh))
        cm = jnp.maximum(cms[0], cms[1])

        @pl.when(i == 0)
        def _():
            mh_ref[0] = cm

        @pl.when(i > 0)
        def _():
            mh_ref[0] = jnp.maximum(mh_ref[0], cm)

    @pl.when(i == NS)
    def _finish():
        mh = mh_ref[0]
        beta = jnp.where(mh > 1.0, 1.0 / mh, 1.0)
        r = yh_ref[...] * beta
        rv = r_ref[...]
        mx2 = jnp.maximum(rv, r)
        mn2 = jnp.minimum(rv, r)
        z = mx2 + GAMMA * jnp.log(1.0 + jnp.exp((mn2 - mx2) * INV_GAMMA))
        mz = jnp.max(z)
        out_ref[...] = jnp.where(mz > 1.0, z / mz, z)


_tc_merge = pl.pallas_call(
    _tc_merge_body,
    grid=(NS + 1,),
    in_specs=[
        pl.BlockSpec((B, G), lambda i: (0, 0)),
        pl.BlockSpec((1, C, NC, PER),
                     lambda i: (jnp.minimum(i, NS - 1), 0, 0, 0)),
        pl.BlockSpec((NW, CL), lambda i: (0, 0)),
        pl.BlockSpec((M, C), lambda i: (0, 0)),
    ],
    out_specs=pl.BlockSpec((B, G), lambda i: (0, 0)),
    out_shape=jax.ShapeDtypeStruct((B, G), jnp.float32),
    scratch_shapes=[
        pltpu.VMEM((B, G), jnp.float32),
        pltpu.SMEM((1,), jnp.float32),
    ],
)


def kernel(x, W, I):
    it2 = (jnp.transpose(I, (0, 2, 3, 1))
           .reshape(C, SL, NS, GSL)
           .transpose(0, 2, 1, 3)
           .reshape(C, NS, CHUNK))
    r = x
    for _ in range(2):
        u = lax.bitcast_convert_type(
            lax.convert_element_type(r, jnp.bfloat16), jnp.uint16)
        pk = lax.bitcast_convert_type(
            u[0::2].astype(jnp.uint32) | (u[1::2].astype(jnp.uint32) << 16),
            jnp.int32)
        y, tmax = _sc_clauses()(pk.reshape(B // 2 * G), it2)
        r = _tc_merge(r, y, tmax, W)
    return r
